# Initial kernel scaffold; baseline (speedup 1.0000x reference)
import jax
import jax.numpy as jnp
from jax.experimental import pallas as pl


def kernel(X, y):
    raise NotImplementedError("write your pallas kernel here")


# speedup vs baseline: 61.6015x; 61.6015x over previous
import functools

import jax
import jax.numpy as jnp
from jax.experimental import pallas as pl
from jax.experimental.pallas import tpu as pltpu
from jax.scipy.special import digamma

_K = 5
_NC = 10
_EPS = 0.01
_MAX_ITER = 50
_INTERPRET = False


def _dist_body(xi_ref, xj_ref, o_ref):
    xi = xi_ref[...]
    xj = xj_ref[...]
    sqi = jnp.sum(xi * xi, axis=1, keepdims=True)
    sqj = jnp.sum(xj * xj, axis=1, keepdims=True)
    dot = jax.lax.dot_general(
        xi, xj, (((1,), (1,)), ((), ())),
        preferred_element_type=jnp.float32)
    d2 = sqi + jnp.transpose(sqj) - 2.0 * dot
    o_ref[...] = jnp.sqrt(jnp.clip(d2, 0.0, None) + 1e-12)


def _pairwise_dists(X, tm=256, tn=256):
    n, d = X.shape
    return pl.pallas_call(
        _dist_body,
        grid=(n // tm, n // tn),
        in_specs=[
            pl.BlockSpec((tm, d), lambda i, j: (i, 0)),
            pl.BlockSpec((tn, d), lambda i, j: (j, 0)),
        ],
        out_specs=pl.BlockSpec((tm, tn), lambda i, j: (i, j)),
        out_shape=jax.ShapeDtypeStruct((n, n), jnp.float32),
        interpret=_INTERPRET,
    )(X, X)


def _stats_body(d_ref, y_ref, yc_ref, o_ref):
    i = pl.program_id(0)
    d = d_ref[...]
    y2 = y_ref[...]
    yr = yc_ref[...]
    lane = jax.lax.broadcasted_iota(jnp.int32, (1, 128), 1)

    dmin_vec = jnp.full((1, 128), jnp.inf, jnp.float32)
    dmax_vec = jnp.full((1, 128), -jnp.inf, jnp.float32)
    cnt_vec = jnp.zeros((1, 128), jnp.float32)
    for c in range(_NC):
        m2 = jnp.logical_and(yr == c, y2 == c)
        dmin_c = jnp.min(jnp.where(m2, d, jnp.inf))
        dmax_c = jnp.max(jnp.where(m2, d, -jnp.inf))
        cnt_c = jnp.sum((y2 == c).astype(jnp.float32))
        dmin_vec = jnp.where(lane == c, dmin_c, dmin_vec)
        dmax_vec = jnp.where(lane == c, dmax_c, dmax_vec)
        cnt_vec = jnp.where(lane == c, cnt_c, cnt_vec)

    pad = jnp.zeros((5, 128), jnp.float32)
    new = jnp.concatenate([dmin_vec, dmax_vec, cnt_vec, pad], axis=0)

    @pl.when(i == 0)
    def _init():
        o_ref[...] = new

    @pl.when(i > 0)
    def _acc():
        prev = o_ref[...]
        o_ref[...] = jnp.concatenate([
            jnp.minimum(prev[0:1, :], dmin_vec),
            jnp.maximum(prev[1:2, :], dmax_vec),
            prev[2:8, :],
        ], axis=0)


def _class_stats(dists, y2, ycol, tr=256):
    n = dists.shape[0]
    return pl.pallas_call(
        _stats_body,
        grid=(n // tr,),
        in_specs=[
            pl.BlockSpec((tr, n), lambda i: (i, 0)),
            pl.BlockSpec((1, n), lambda i: (0, 0)),
            pl.BlockSpec((tr, 1), lambda i: (i, 0)),
        ],
        out_specs=pl.BlockSpec((8, 128), lambda i: (0, 0)),
        out_shape=jax.ShapeDtypeStruct((8, 128), jnp.float32),
        interpret=_INTERPRET,
    )(dists, y2, ycol)


def _sinkhorn_body(p_ref, cs_ref, d_ref, y_ref, yc_ref, o_ref, *, tr, w):
    i = pl.program_id(0)
    sk = pl.multiple_of(cs_ref[0, i], 128)
    dw = d_ref[:, pl.ds(sk, w)]
    yw = y_ref[:, pl.ds(sk, w)]
    yr = yc_ref[...]
    mask = yw == yr

    def sel(row):
        v = jnp.zeros((tr, 1), jnp.float32)
        for c in range(_NC):
            v = jnp.where(yr == c, p_ref[row, c], v)
        return v

    smin = sel(0)
    inv_den = sel(1)
    log_nu0 = sel(2)
    log_nu1 = sel(3)
    log_mu = sel(4)

    inv_eps = 1.0 / _EPS
    s = jnp.log(1.0 / (dw + 1e-6))
    sn = (s - smin) * inv_den
    c0i = sn * sn * inv_eps
    c1i = (sn - 1.0) * (sn - 1.0) * inv_eps
    f0 = jnp.where(mask, 0.0, -jnp.inf)

    def body(_, carry):
        fi, _g0, _g1 = carry
        t0 = fi - c0i
        t1 = fi - c1i
        m0 = jnp.max(t0, axis=1, keepdims=True)
        m1 = jnp.max(t1, axis=1, keepdims=True)
        g0i = log_nu0 - (m0 + jnp.log(jnp.sum(jnp.exp(t0 - m0), axis=1,
                                              keepdims=True)))
        g1i = log_nu1 - (m1 + jnp.log(jnp.sum(jnp.exp(t1 - m1), axis=1,
                                              keepdims=True)))
        u0 = g0i - c0i
        u1 = g1i - c1i
        mm = jnp.maximum(u0, u1)
        lse = mm + jnp.log1p(jnp.exp(-jnp.abs(u1 - u0)))
        fi = log_mu - lse
        fi = jnp.where(mask, fi, -jnp.inf)
        return fi, g0i, g1i

    zero = jnp.zeros((tr, 1), jnp.float32)
    fi, _, g1i = jax.lax.fori_loop(0, _MAX_ITER, body, (f0, zero, zero))

    wgt = jnp.exp(fi + g1i - c1i)
    anchor = jnp.sum(jnp.where(mask, dw * wgt, 0.0), axis=1, keepdims=True)
    hard = jnp.where(anchor - d_ref[...] >= 0.0, 1.0, 0.0)
    o_ref[...] = jnp.sum(hard, axis=1, keepdims=True)


def _sinkhorn_counts(dists, y2, ycol, params, cstarts, w, tr=128):
    n = dists.shape[0]
    return pl.pallas_call(
        functools.partial(_sinkhorn_body, tr=tr, w=w),
        grid=(n // tr,),
        in_specs=[
            pl.BlockSpec(memory_space=pltpu.SMEM),
            pl.BlockSpec(memory_space=pltpu.SMEM),
            pl.BlockSpec((tr, n), lambda i: (i, 0)),
            pl.BlockSpec((1, n), lambda i: (0, 0)),
            pl.BlockSpec((tr, 1), lambda i: (i, 0)),
        ],
        out_specs=pl.BlockSpec((tr, 1), lambda i: (i, 0)),
        out_shape=jax.ShapeDtypeStruct((n, 1), jnp.float32),
        interpret=_INTERPRET,
    )(params, cstarts, dists, y2, ycol)


def kernel(X, y):
    n = X.shape[0]
    tr = min(128, n)
    w = min(1024, n)

    perm = jnp.argsort(y)
    yp = y[perm]
    Xp = X[perm]
    y2 = jnp.reshape(yp, (1, n))
    ycol = jnp.reshape(yp, (n, 1))

    dists = _pairwise_dists(Xp)
    stats = _class_stats(dists, y2, ycol)

    dmin = stats[0, :_NC]
    dmax = stats[1, :_NC]
    cnt = stats[2, :_NC]

    smax = jnp.log(1.0 / (dmin + 1e-6))
    smin = jnp.log(1.0 / (dmax + 1e-6))
    inv_den = 1.0 / (smax - smin + 1e-8)
    kk = float(_K + 1)
    log_nu0 = jnp.log((cnt - kk) / cnt)
    log_nu1 = jnp.log(kk / cnt)
    log_mu = -jnp.log(cnt)

    params = jnp.zeros((8, 16), jnp.float32)
    params = params.at[0, :_NC].set(smin)
    params = params.at[1, :_NC].set(inv_den)
    params = params.at[2, :_NC].set(log_nu0)
    params = params.at[3, :_NC].set(log_nu1)
    params = params.at[4, :_NC].set(log_mu)

    cnt_i = cnt.astype(jnp.int32)
    starts = jnp.concatenate([jnp.zeros((1,), jnp.int32),
                              jnp.cumsum(cnt_i)[:-1]])
    ends = starts + cnt_i
    c_lo = yp[0::tr]
    c_hi = yp[tr - 1::tr]
    win_lo = starts[c_lo]
    win_hi = ends[c_hi]
    cstart = jnp.clip((win_lo // 128) * 128, 0, n - w)
    fits = jnp.all(win_hi - cstart <= w)
    cstarts = jnp.reshape(cstart, (1, n // tr))
    zeros = jnp.zeros_like(cstarts)

    cnts = jax.lax.cond(
        fits,
        lambda: _sinkhorn_counts(dists, y2, ycol, params, cstarts, w, tr),
        lambda: _sinkhorn_counts(dists, y2, ycol, params, zeros, n, tr),
    )[:, 0]

    m_shift = jnp.float32(-1.0) + jnp.float32(1e-7)
    avg_m_i = jnp.mean(digamma(cnts + m_shift))
    n_x_w = cnt / float(n)
    avg_n_x = jnp.sum(n_x_w * digamma(cnt))
    mi = (digamma(jnp.asarray(float(n), jnp.float32)) - avg_n_x
          + digamma(jnp.asarray(float(_K), jnp.float32)) - avg_m_i)
    mi = mi / jnp.log(jnp.asarray(2.0, jnp.float32))
    return jax.nn.relu(mi)


# speedup vs baseline: 76.0583x; 1.2347x over previous
import functools

import jax
import jax.numpy as jnp
from jax.experimental import pallas as pl
from jax.experimental.pallas import tpu as pltpu
from jax.scipy.special import digamma

_K = 5
_NC = 10
_EPS = 0.01
_MAX_ITER = 50
_INTERPRET = False


def _dist_body(xi_ref, xj_ref, o_ref):
    xi = xi_ref[...]
    xj = xj_ref[...]
    sqi = jnp.sum(xi * xi, axis=1, keepdims=True)
    sqj = jnp.sum(xj * xj, axis=1, keepdims=True)
    dot = jax.lax.dot_general(
        xi, xj, (((1,), (1,)), ((), ())),
        preferred_element_type=jnp.float32)
    d2 = sqi + jnp.transpose(sqj) - 2.0 * dot
    o_ref[...] = jnp.sqrt(jnp.clip(d2, 0.0, None) + 1e-12)


def _pairwise_dists(X, tm=256, tn=256):
    n, d = X.shape
    return pl.pallas_call(
        _dist_body,
        grid=(n // tm, n // tn),
        in_specs=[
            pl.BlockSpec((tm, d), lambda i, j: (i, 0)),
            pl.BlockSpec((tn, d), lambda i, j: (j, 0)),
        ],
        out_specs=pl.BlockSpec((tm, tn), lambda i, j: (i, j)),
        out_shape=jax.ShapeDtypeStruct((n, n), jnp.float32),
        interpret=_INTERPRET,
    )(X, X)


def _stats_body(bnd_ref, cs_ref, d_ref, y_ref, yc_ref, o_ref, *, tr, w):
    i = pl.program_id(0)
    sk = pl.multiple_of(cs_ref[0, i], 128)
    dw = d_ref[:, pl.ds(sk, w)]
    yw = y_ref[:, pl.ds(sk, w)]
    yr = yc_ref[...]
    c_lo = bnd_ref[0, i]
    c_hi = bnd_ref[1, i]
    lane = jax.lax.broadcasted_iota(jnp.int32, (1, 128), 1)

    @pl.when(i == 0)
    def _init():
        row = jax.lax.broadcasted_iota(jnp.int32, (8, 128), 0)
        o_ref[...] = jnp.where(row == 0, jnp.inf,
                               jnp.where(row == 1, -jnp.inf, 0.0)
                               ).astype(jnp.float32)

    for c in range(_NC):
        @pl.when(jnp.logical_and(c >= c_lo, c <= c_hi))
        def _acc(c=c):
            m2 = jnp.logical_and(yr == c, yw == c)
            dmin_c = jnp.min(jnp.where(m2, dw, jnp.inf))
            dmax_c = jnp.max(jnp.where(m2, dw, -jnp.inf))
            p0 = o_ref[0:1, :]
            o_ref[0:1, :] = jnp.where(lane == c,
                                      jnp.minimum(p0, dmin_c), p0)
            p1 = o_ref[1:2, :]
            o_ref[1:2, :] = jnp.where(lane == c,
                                      jnp.maximum(p1, dmax_c), p1)


def _class_stats(dists, y2, ycol, bounds, cstarts, w, tr=128):
    n = dists.shape[0]
    return pl.pallas_call(
        functools.partial(_stats_body, tr=tr, w=w),
        grid=(n // tr,),
        in_specs=[
            pl.BlockSpec(memory_space=pltpu.SMEM),
            pl.BlockSpec(memory_space=pltpu.SMEM),
            pl.BlockSpec((tr, n), lambda i: (i, 0)),
            pl.BlockSpec((1, n), lambda i: (0, 0)),
            pl.BlockSpec((tr, 1), lambda i: (i, 0)),
        ],
        out_specs=pl.BlockSpec((8, 128), lambda i: (0, 0)),
        out_shape=jax.ShapeDtypeStruct((8, 128), jnp.float32),
        interpret=_INTERPRET,
    )(bounds, cstarts, dists, y2, ycol)


def _sinkhorn_body(p_ref, cs_ref, d_ref, y_ref, yc_ref, o_ref, *, tr, w):
    i = pl.program_id(0)
    sk = pl.multiple_of(cs_ref[0, i], 128)
    dw = d_ref[:, pl.ds(sk, w)]
    yw = y_ref[:, pl.ds(sk, w)]
    yr = yc_ref[...]
    mask = yw == yr

    def sel(row):
        v = jnp.zeros((tr, 1), jnp.float32)
        for c in range(_NC):
            v = jnp.where(yr == c, p_ref[row, c], v)
        return v

    smin = sel(0)
    inv_den = sel(1)
    log_nu0 = sel(2)
    log_nu1 = sel(3)
    log_mu = sel(4)

    inv_eps = 1.0 / _EPS
    s = jnp.log(1.0 / (dw + 1e-6))
    sn = (s - smin) * inv_den
    c0i = sn * sn * inv_eps
    c1i = (sn - 1.0) * (sn - 1.0) * inv_eps
    f0 = jnp.where(mask, 0.0, -jnp.inf)

    def body(_, carry):
        fi, _g0, _g1 = carry
        t0 = fi - c0i
        t1 = fi - c1i
        m0 = jnp.max(t0, axis=1, keepdims=True)
        m1 = jnp.max(t1, axis=1, keepdims=True)
        g0i = log_nu0 - (m0 + jnp.log(jnp.sum(jnp.exp(t0 - m0), axis=1,
                                              keepdims=True)))
        g1i = log_nu1 - (m1 + jnp.log(jnp.sum(jnp.exp(t1 - m1), axis=1,
                                              keepdims=True)))
        u0 = g0i - c0i
        u1 = g1i - c1i
        mm = jnp.maximum(u0, u1)
        lse = mm + jnp.log1p(jnp.exp(-jnp.abs(u1 - u0)))
        fi = log_mu - lse
        fi = jnp.where(mask, fi, -jnp.inf)
        return fi, g0i, g1i

    zero = jnp.zeros((tr, 1), jnp.float32)
    fi, _, g1i = jax.lax.fori_loop(0, _MAX_ITER, body, (f0, zero, zero))

    wgt = jnp.exp(fi + g1i - c1i)
    anchor = jnp.sum(jnp.where(mask, dw * wgt, 0.0), axis=1, keepdims=True)
    hard = jnp.where(anchor - d_ref[...] >= 0.0, 1.0, 0.0)
    o_ref[...] = jnp.sum(hard, axis=1, keepdims=True)


def _sinkhorn_counts(dists, y2, ycol, params, cstarts, w, tr=128):
    n = dists.shape[0]
    return pl.pallas_call(
        functools.partial(_sinkhorn_body, tr=tr, w=w),
        grid=(n // tr,),
        in_specs=[
            pl.BlockSpec(memory_space=pltpu.SMEM),
            pl.BlockSpec(memory_space=pltpu.SMEM),
            pl.BlockSpec((tr, n), lambda i: (i, 0)),
            pl.BlockSpec((1, n), lambda i: (0, 0)),
            pl.BlockSpec((tr, 1), lambda i: (i, 0)),
        ],
        out_specs=pl.BlockSpec((tr, 1), lambda i: (i, 0)),
        out_shape=jax.ShapeDtypeStruct((n, 1), jnp.float32),
        interpret=_INTERPRET,
    )(params, cstarts, dists, y2, ycol)


def kernel(X, y):
    n = X.shape[0]
    tr = min(128, n)
    w = min(768, n)

    perm = jnp.argsort(y)
    yp = y[perm]
    Xp = X[perm]
    y2 = jnp.reshape(yp, (1, n))
    ycol = jnp.reshape(yp, (n, 1))

    cnt_i = jnp.sum((y[None, :] == jnp.arange(_NC, dtype=y.dtype)[:, None])
                    .astype(jnp.int32), axis=1)
    cnt = cnt_i.astype(jnp.float32)
    starts = jnp.concatenate([jnp.zeros((1,), jnp.int32),
                              jnp.cumsum(cnt_i)[:-1]])
    ends = starts + cnt_i
    c_lo = yp[0::tr].astype(jnp.int32)
    c_hi = yp[tr - 1::tr].astype(jnp.int32)
    win_lo = starts[c_lo]
    win_hi = ends[c_hi]
    cstart = jnp.clip((win_lo // 128) * 128, 0, n - w)
    fits = jnp.all(win_hi - cstart <= w)
    nb = n // tr
    cstarts = jnp.reshape(cstart, (1, nb))
    zeros = jnp.zeros_like(cstarts)
    bounds = jnp.stack([c_lo, c_hi])
    bounds_full = jnp.stack([jnp.zeros((nb,), jnp.int32),
                             jnp.full((nb,), _NC - 1, jnp.int32)])

    dists = _pairwise_dists(Xp)

    stats = jax.lax.cond(
        fits,
        lambda: _class_stats(dists, y2, ycol, bounds, cstarts, w, tr),
        lambda: _class_stats(dists, y2, ycol, bounds_full, zeros, n, tr),
    )

    dmin = stats[0, :_NC]
    dmax = stats[1, :_NC]

    smax = jnp.log(1.0 / (dmin + 1e-6))
    smin = jnp.log(1.0 / (dmax + 1e-6))
    inv_den = 1.0 / (smax - smin + 1e-8)
    kk = float(_K + 1)
    log_nu0 = jnp.log((cnt - kk) / cnt)
    log_nu1 = jnp.log(kk / cnt)
    log_mu = -jnp.log(cnt)

    params = jnp.zeros((8, 16), jnp.float32)
    params = params.at[0, :_NC].set(smin)
    params = params.at[1, :_NC].set(inv_den)
    params = params.at[2, :_NC].set(log_nu0)
    params = params.at[3, :_NC].set(log_nu1)
    params = params.at[4, :_NC].set(log_mu)

    cnts = jax.lax.cond(
        fits,
        lambda: _sinkhorn_counts(dists, y2, ycol, params, cstarts, w, tr),
        lambda: _sinkhorn_counts(dists, y2, ycol, params, zeros, n, tr),
    )[:, 0]

    m_shift = jnp.float32(-1.0) + jnp.float32(1e-7)
    avg_m_i = jnp.mean(digamma(cnts + m_shift))
    n_x_w = cnt / float(n)
    avg_n_x = jnp.sum(n_x_w * digamma(cnt))
    mi = (digamma(jnp.asarray(float(n), jnp.float32)) - avg_n_x
          + digamma(jnp.asarray(float(_K), jnp.float32)) - avg_m_i)
    mi = mi / jnp.log(jnp.asarray(2.0, jnp.float32))
    return jax.nn.relu(mi)


# speedup vs baseline: 79.5922x; 1.0465x over previous
import functools

import jax
import jax.numpy as jnp
from jax.experimental import pallas as pl
from jax.experimental.pallas import tpu as pltpu
from jax.scipy.special import digamma

_K = 5
_NC = 10
_EPS = 0.01
_MAX_ITER = 50
_INTERPRET = False


def _dist_body(xi_ref, xj_ref, o_ref):
    xi = xi_ref[...]
    xj = xj_ref[...]
    sqi = jnp.sum(xi * xi, axis=1, keepdims=True)
    sqj = jnp.sum(xj * xj, axis=1, keepdims=True)
    dot = jax.lax.dot_general(
        xi, xj, (((1,), (1,)), ((), ())),
        preferred_element_type=jnp.float32)
    d2 = sqi + jnp.transpose(sqj) - 2.0 * dot
    o_ref[...] = jnp.sqrt(jnp.clip(d2, 0.0, None) + 1e-12)


def _pairwise_dists(X, tm=512):
    n, d = X.shape
    tm = min(tm, n)
    return pl.pallas_call(
        _dist_body,
        grid=(n // tm,),
        in_specs=[
            pl.BlockSpec((tm, d), lambda i: (i, 0)),
            pl.BlockSpec((n, d), lambda i: (0, 0)),
        ],
        out_specs=pl.BlockSpec((tm, n), lambda i: (i, 0)),
        out_shape=jax.ShapeDtypeStruct((n, n), jnp.float32),
        interpret=_INTERPRET,
    )(X, X)


def _stats_body(bnd_ref, cs_ref, d_ref, y_ref, yc_ref, o_ref, *, tr, w):
    i = pl.program_id(0)
    sk = pl.multiple_of(cs_ref[0, i], 128)
    dw = d_ref[:, pl.ds(sk, w)]
    yw = y_ref[:, pl.ds(sk, w)]
    yr = yc_ref[...]
    c_lo = bnd_ref[0, i]
    c_hi = bnd_ref[1, i]
    lane = jax.lax.broadcasted_iota(jnp.int32, (1, 128), 1)

    @pl.when(i == 0)
    def _init():
        row = jax.lax.broadcasted_iota(jnp.int32, (8, 128), 0)
        o_ref[...] = jnp.where(row == 0, jnp.inf,
                               jnp.where(row == 1, -jnp.inf, 0.0)
                               ).astype(jnp.float32)

    for c in range(_NC):
        @pl.when(jnp.logical_and(c >= c_lo, c <= c_hi))
        def _acc(c=c):
            m2 = jnp.logical_and(yr == c, yw == c)
            dmin_c = jnp.min(jnp.where(m2, dw, jnp.inf))
            dmax_c = jnp.max(jnp.where(m2, dw, -jnp.inf))
            p0 = o_ref[0:1, :]
            o_ref[0:1, :] = jnp.where(lane == c,
                                      jnp.minimum(p0, dmin_c), p0)
            p1 = o_ref[1:2, :]
            o_ref[1:2, :] = jnp.where(lane == c,
                                      jnp.maximum(p1, dmax_c), p1)


def _class_stats(dists, y2, ycol, bounds, cstarts, w, tr=128):
    n = dists.shape[0]
    return pl.pallas_call(
        functools.partial(_stats_body, tr=tr, w=w),
        grid=(n // tr,),
        in_specs=[
            pl.BlockSpec(memory_space=pltpu.SMEM),
            pl.BlockSpec(memory_space=pltpu.SMEM),
            pl.BlockSpec((tr, n), lambda i: (i, 0)),
            pl.BlockSpec((1, n), lambda i: (0, 0)),
            pl.BlockSpec((tr, 1), lambda i: (i, 0)),
        ],
        out_specs=pl.BlockSpec((8, 128), lambda i: (0, 0)),
        out_shape=jax.ShapeDtypeStruct((8, 128), jnp.float32),
        interpret=_INTERPRET,
    )(bounds, cstarts, dists, y2, ycol)


def _sinkhorn_body(p_ref, cs_ref, d_ref, y_ref, yc_ref, o_ref, *, tr, w):
    i = pl.program_id(0)
    sk = pl.multiple_of(cs_ref[0, i], 128)
    dw = d_ref[:, pl.ds(sk, w)]
    yw = y_ref[:, pl.ds(sk, w)]
    yr = yc_ref[...]
    mask = yw == yr

    def sel(row):
        v = jnp.zeros((tr, 1), jnp.float32)
        for c in range(_NC):
            v = jnp.where(yr == c, p_ref[row, c], v)
        return v

    smin = sel(0)
    inv_den = sel(1)
    log_nu0 = sel(2)
    log_nu1 = sel(3)
    log_mu = sel(4)

    inv_eps = 1.0 / _EPS
    s = jnp.log(1.0 / (dw + 1e-6))
    sn = (s - smin) * inv_den
    c0i = sn * sn * inv_eps
    c1i = (sn - 1.0) * (sn - 1.0) * inv_eps
    f0 = jnp.where(mask, 0.0, -jnp.inf)

    def body(_, carry):
        fi, _g0, _g1 = carry
        t0 = fi - c0i
        t1 = fi - c1i
        m0 = jnp.max(t0, axis=1, keepdims=True)
        m1 = jnp.max(t1, axis=1, keepdims=True)
        g0i = log_nu0 - (m0 + jnp.log(jnp.sum(jnp.exp(t0 - m0), axis=1,
                                              keepdims=True)))
        g1i = log_nu1 - (m1 + jnp.log(jnp.sum(jnp.exp(t1 - m1), axis=1,
                                              keepdims=True)))
        u0 = g0i - c0i
        u1 = g1i - c1i
        mm = jnp.maximum(u0, u1)
        lse = mm + jnp.log1p(jnp.exp(-jnp.abs(u1 - u0)))
        fi = log_mu - lse
        fi = jnp.where(mask, fi, -jnp.inf)
        return fi, g0i, g1i

    zero = jnp.zeros((tr, 1), jnp.float32)
    fi, _, g1i = jax.lax.fori_loop(0, _MAX_ITER, body, (f0, zero, zero))

    wgt = jnp.exp(fi + g1i - c1i)
    anchor = jnp.sum(jnp.where(mask, dw * wgt, 0.0), axis=1, keepdims=True)
    hard = jnp.where(anchor - d_ref[...] >= 0.0, 1.0, 0.0)
    o_ref[...] = jnp.sum(hard, axis=1, keepdims=True)


def _sinkhorn_counts(dists, y2, ycol, params, cstarts, w, tr=128):
    n = dists.shape[0]
    return pl.pallas_call(
        functools.partial(_sinkhorn_body, tr=tr, w=w),
        grid=(n // tr,),
        in_specs=[
            pl.BlockSpec(memory_space=pltpu.SMEM),
            pl.BlockSpec(memory_space=pltpu.SMEM),
            pl.BlockSpec((tr, n), lambda i: (i, 0)),
            pl.BlockSpec((1, n), lambda i: (0, 0)),
            pl.BlockSpec((tr, 1), lambda i: (i, 0)),
        ],
        out_specs=pl.BlockSpec((tr, 1), lambda i: (i, 0)),
        out_shape=jax.ShapeDtypeStruct((n, 1), jnp.float32),
        interpret=_INTERPRET,
    )(params, cstarts, dists, y2, ycol)


def kernel(X, y):
    n = X.shape[0]
    tr = min(128, n)
    w = min(768, n)

    oh = (y[:, None] == jnp.arange(_NC, dtype=y.dtype)[None, :]
          ).astype(jnp.int32)
    cnt_i = jnp.sum(oh, axis=0)
    cnt = cnt_i.astype(jnp.float32)
    starts = jnp.concatenate([jnp.zeros((1,), jnp.int32),
                              jnp.cumsum(cnt_i)[:-1]])
    ends = starts + cnt_i
    within = jnp.cumsum(oh, axis=0) - oh
    rank = starts[y] + jnp.sum(within * oh, axis=1)
    perm = jnp.zeros((n,), jnp.int32).at[rank].set(
        jnp.arange(n, dtype=jnp.int32))
    yp = y[perm]
    Xp = X[perm]
    y2 = jnp.reshape(yp, (1, n))
    ycol = jnp.reshape(yp, (n, 1))
    c_lo = yp[0::tr].astype(jnp.int32)
    c_hi = yp[tr - 1::tr].astype(jnp.int32)
    win_lo = starts[c_lo]
    win_hi = ends[c_hi]
    cstart = jnp.clip((win_lo // 128) * 128, 0, n - w)
    fits = jnp.all(win_hi - cstart <= w)
    nb = n // tr
    cstarts = jnp.reshape(cstart, (1, nb))
    zeros = jnp.zeros_like(cstarts)
    bounds = jnp.stack([c_lo, c_hi])
    bounds_full = jnp.stack([jnp.zeros((nb,), jnp.int32),
                             jnp.full((nb,), _NC - 1, jnp.int32)])

    dists = _pairwise_dists(Xp)

    stats = jax.lax.cond(
        fits,
        lambda: _class_stats(dists, y2, ycol, bounds, cstarts, w, tr),
        lambda: _class_stats(dists, y2, ycol, bounds_full, zeros, n, tr),
    )

    dmin = stats[0, :_NC]
    dmax = stats[1, :_NC]

    smax = jnp.log(1.0 / (dmin + 1e-6))
    smin = jnp.log(1.0 / (dmax + 1e-6))
    inv_den = 1.0 / (smax - smin + 1e-8)
    kk = float(_K + 1)
    log_nu0 = jnp.log((cnt - kk) / cnt)
    log_nu1 = jnp.log(kk / cnt)
    log_mu = -jnp.log(cnt)

    params = jnp.zeros((8, 16), jnp.float32)
    params = params.at[0, :_NC].set(smin)
    params = params.at[1, :_NC].set(inv_den)
    params = params.at[2, :_NC].set(log_nu0)
    params = params.at[3, :_NC].set(log_nu1)
    params = params.at[4, :_NC].set(log_mu)

    cnts = jax.lax.cond(
        fits,
        lambda: _sinkhorn_counts(dists, y2, ycol, params, cstarts, w, tr),
        lambda: _sinkhorn_counts(dists, y2, ycol, params, zeros, n, tr),
    )[:, 0]

    m_shift = jnp.float32(-1.0) + jnp.float32(1e-7)
    avg_m_i = jnp.mean(digamma(cnts + m_shift))
    n_x_w = cnt / float(n)
    avg_n_x = jnp.sum(n_x_w * digamma(cnt))
    mi = (digamma(jnp.asarray(float(n), jnp.float32)) - avg_n_x
          + digamma(jnp.asarray(float(_K), jnp.float32)) - avg_m_i)
    mi = mi / jnp.log(jnp.asarray(2.0, jnp.float32))
    return jax.nn.relu(mi)


# speedup vs baseline: 86.0749x; 1.0814x over previous
import functools

import jax
import jax.numpy as jnp
from jax.experimental import pallas as pl
from jax.experimental.pallas import tpu as pltpu
from jax.scipy.special import digamma

_K = 5
_NC = 10
_EPS = 0.01
_MAX_ITER = 50
_INTERPRET = False


def _dist_body(xi_ref, xj_ref, o_ref):
    xi = xi_ref[...]
    xj = xj_ref[...]
    sqi = jnp.sum(xi * xi, axis=1, keepdims=True)
    sqj = jnp.sum(xj * xj, axis=1, keepdims=True)
    dot = jax.lax.dot_general(
        xi, xj, (((1,), (1,)), ((), ())),
        preferred_element_type=jnp.float32)
    d2 = sqi + jnp.transpose(sqj) - 2.0 * dot
    o_ref[...] = jnp.sqrt(jnp.clip(d2, 0.0, None) + 1e-12)


def _pairwise_dists(X, tm=512):
    n, d = X.shape
    tm = min(tm, n)
    return pl.pallas_call(
        _dist_body,
        grid=(n // tm,),
        in_specs=[
            pl.BlockSpec((tm, d), lambda i: (i, 0)),
            pl.BlockSpec((n, d), lambda i: (0, 0)),
        ],
        out_specs=pl.BlockSpec((tm, n), lambda i: (i, 0)),
        out_shape=jax.ShapeDtypeStruct((n, n), jnp.float32),
        interpret=_INTERPRET,
    )(X, X)


def _stats_body(bnd_ref, cs_ref, d_ref, y_ref, yc_ref, o_ref, *, tr, w):
    i = pl.program_id(0)
    sk = pl.multiple_of(cs_ref[0, i], 128)
    dw = d_ref[:, pl.ds(sk, w)]
    yw = y_ref[:, pl.ds(sk, w)]
    yr = yc_ref[...]
    c_lo = bnd_ref[0, i]
    c_hi = bnd_ref[1, i]
    lane = jax.lax.broadcasted_iota(jnp.int32, (1, 128), 1)

    @pl.when(i == 0)
    def _init():
        row = jax.lax.broadcasted_iota(jnp.int32, (8, 128), 0)
        o_ref[...] = jnp.where(row == 0, jnp.inf,
                               jnp.where(row == 1, -jnp.inf, 0.0)
                               ).astype(jnp.float32)

    for c in range(_NC):
        @pl.when(jnp.logical_and(c >= c_lo, c <= c_hi))
        def _acc(c=c):
            m2 = jnp.logical_and(yr == c, yw == c)
            dmin_c = jnp.min(jnp.where(m2, dw, jnp.inf))
            dmax_c = jnp.max(jnp.where(m2, dw, -jnp.inf))
            p0 = o_ref[0:1, :]
            o_ref[0:1, :] = jnp.where(lane == c,
                                      jnp.minimum(p0, dmin_c), p0)
            p1 = o_ref[1:2, :]
            o_ref[1:2, :] = jnp.where(lane == c,
                                      jnp.maximum(p1, dmax_c), p1)


def _class_stats(dists, y2, ycol, bounds, cstarts, w, tr=128):
    n = dists.shape[0]
    return pl.pallas_call(
        functools.partial(_stats_body, tr=tr, w=w),
        grid=(n // tr,),
        in_specs=[
            pl.BlockSpec(memory_space=pltpu.SMEM),
            pl.BlockSpec(memory_space=pltpu.SMEM),
            pl.BlockSpec((tr, n), lambda i: (i, 0)),
            pl.BlockSpec((1, n), lambda i: (0, 0)),
            pl.BlockSpec((tr, 1), lambda i: (i, 0)),
        ],
        out_specs=pl.BlockSpec((8, 128), lambda i: (0, 0)),
        out_shape=jax.ShapeDtypeStruct((8, 128), jnp.float32),
        interpret=_INTERPRET,
    )(bounds, cstarts, dists, y2, ycol)


def _sinkhorn_body(p_ref, cs_ref, d_ref, y_ref, yc_ref, o_ref, *, tr, w):
    i = pl.program_id(0)
    sk = pl.multiple_of(cs_ref[0, i], 128)
    dw = d_ref[:, pl.ds(sk, w)]
    yw = y_ref[:, pl.ds(sk, w)]
    yr = yc_ref[...]
    mask = yw == yr

    def sel(row):
        v = jnp.zeros((tr, 1), jnp.float32)
        for c in range(_NC):
            v = jnp.where(yr == c, p_ref[row, c], v)
        return v

    smin = sel(0)
    inv_den = sel(1)
    log_nu0 = sel(2)
    log_nu1 = sel(3)
    log_mu = sel(4)

    inv_eps = 1.0 / _EPS
    s = jnp.log(1.0 / (dw + 1e-6))
    sn = (s - smin) * inv_den
    c0i = sn * sn * inv_eps
    c1i = (sn - 1.0) * (sn - 1.0) * inv_eps
    f0 = jnp.where(mask, 0.0, -jnp.inf)

    def body(_, carry):
        fi, _g0, _g1 = carry
        t0 = fi - c0i
        t1 = fi - c1i
        m0 = jnp.max(t0, axis=1, keepdims=True)
        m1 = jnp.max(t1, axis=1, keepdims=True)
        g0i = log_nu0 - (m0 + jnp.log(jnp.sum(jnp.exp(t0 - m0), axis=1,
                                              keepdims=True)))
        g1i = log_nu1 - (m1 + jnp.log(jnp.sum(jnp.exp(t1 - m1), axis=1,
                                              keepdims=True)))
        u0 = g0i - c0i
        u1 = g1i - c1i
        mm = jnp.maximum(u0, u1)
        lse = mm + jnp.log1p(jnp.exp(-jnp.abs(u1 - u0)))
        fi = log_mu - lse
        fi = jnp.where(mask, fi, -jnp.inf)
        return fi, g0i, g1i

    zero = jnp.zeros((tr, 1), jnp.float32)
    fi, _, g1i = jax.lax.fori_loop(0, _MAX_ITER, body, (f0, zero, zero))

    wgt = jnp.exp(fi + g1i - c1i)
    anchor = jnp.sum(jnp.where(mask, dw * wgt, 0.0), axis=1, keepdims=True)
    hard = jnp.where(anchor - d_ref[...] >= 0.0, 1.0, 0.0)
    o_ref[...] = jnp.sum(hard, axis=1, keepdims=True)


def _sinkhorn_counts(dists, y2, ycol, params, cstarts, w, tr=128):
    n = dists.shape[0]
    return pl.pallas_call(
        functools.partial(_sinkhorn_body, tr=tr, w=w),
        grid=(n // tr,),
        in_specs=[
            pl.BlockSpec(memory_space=pltpu.SMEM),
            pl.BlockSpec(memory_space=pltpu.SMEM),
            pl.BlockSpec((tr, n), lambda i: (i, 0)),
            pl.BlockSpec((1, n), lambda i: (0, 0)),
            pl.BlockSpec((tr, 1), lambda i: (i, 0)),
        ],
        out_specs=pl.BlockSpec((tr, 1), lambda i: (i, 0)),
        out_shape=jax.ShapeDtypeStruct((n, 1), jnp.float32),
        interpret=_INTERPRET,
    )(params, cstarts, dists, y2, ycol)


def kernel(X, y):
    n = X.shape[0]
    tr = min(128, n)
    w = min(640, n)

    oh = (y[:, None] == jnp.arange(_NC, dtype=y.dtype)[None, :]
          ).astype(jnp.int32)
    cnt_i = jnp.sum(oh, axis=0)
    cnt = cnt_i.astype(jnp.float32)
    starts = jnp.concatenate([jnp.zeros((1,), jnp.int32),
                              jnp.cumsum(cnt_i)[:-1]])
    ends = starts + cnt_i
    within = jnp.cumsum(oh, axis=0) - oh
    rank = starts[y] + jnp.sum(within * oh, axis=1)
    perm = jnp.zeros((n,), jnp.int32).at[rank].set(
        jnp.arange(n, dtype=jnp.int32))
    yp = y[perm]
    Xp = X[perm]
    y2 = jnp.reshape(yp, (1, n))
    ycol = jnp.reshape(yp, (n, 1))
    c_lo = yp[0::tr].astype(jnp.int32)
    c_hi = yp[tr - 1::tr].astype(jnp.int32)
    win_lo = starts[c_lo]
    win_hi = ends[c_hi]
    cstart = jnp.clip((win_lo // 128) * 128, 0, n - w)
    fits = jnp.all(win_hi - cstart <= w)
    nb = n // tr
    cstarts = jnp.reshape(cstart, (1, nb))
    zeros = jnp.zeros_like(cstarts)
    bounds = jnp.stack([c_lo, c_hi])
    bounds_full = jnp.stack([jnp.zeros((nb,), jnp.int32),
                             jnp.full((nb,), _NC - 1, jnp.int32)])

    dists = _pairwise_dists(Xp)

    stats = jax.lax.cond(
        fits,
        lambda: _class_stats(dists, y2, ycol, bounds, cstarts, w, tr),
        lambda: _class_stats(dists, y2, ycol, bounds_full, zeros, n, tr),
    )

    dmin = stats[0, :_NC]
    dmax = stats[1, :_NC]

    smax = jnp.log(1.0 / (dmin + 1e-6))
    smin = jnp.log(1.0 / (dmax + 1e-6))
    inv_den = 1.0 / (smax - smin + 1e-8)
    kk = float(_K + 1)
    log_nu0 = jnp.log((cnt - kk) / cnt)
    log_nu1 = jnp.log(kk / cnt)
    log_mu = -jnp.log(cnt)

    params = jnp.zeros((8, 16), jnp.float32)
    params = params.at[0, :_NC].set(smin)
    params = params.at[1, :_NC].set(inv_den)
    params = params.at[2, :_NC].set(log_nu0)
    params = params.at[3, :_NC].set(log_nu1)
    params = params.at[4, :_NC].set(log_mu)

    cnts = jax.lax.cond(
        fits,
        lambda: _sinkhorn_counts(dists, y2, ycol, params, cstarts, w, tr),
        lambda: _sinkhorn_counts(dists, y2, ycol, params, zeros, n, tr),
    )[:, 0]

    m_shift = jnp.float32(-1.0) + jnp.float32(1e-7)
    avg_m_i = jnp.mean(digamma(cnts + m_shift))
    n_x_w = cnt / float(n)
    avg_n_x = jnp.sum(n_x_w * digamma(cnt))
    mi = (digamma(jnp.asarray(float(n), jnp.float32)) - avg_n_x
          + digamma(jnp.asarray(float(_K), jnp.float32)) - avg_m_i)
    mi = mi / jnp.log(jnp.asarray(2.0, jnp.float32))
    return jax.nn.relu(mi)


# speedup vs baseline: 88.0676x; 1.0232x over previous
import functools

import jax
import jax.numpy as jnp
from jax.experimental import pallas as pl
from jax.experimental.pallas import tpu as pltpu
from jax.scipy.special import digamma

_K = 5
_NC = 10
_EPS = 0.01
_MAX_ITER = 50
_INTERPRET = False


def _dist_body(xi_ref, xj_ref, o_ref):
    xi = xi_ref[...]
    xj = xj_ref[...]
    sqi = jnp.sum(xi * xi, axis=1, keepdims=True)
    sqj = jnp.sum(xj * xj, axis=1, keepdims=True)
    dot = jax.lax.dot_general(
        xi, xj, (((1,), (1,)), ((), ())),
        preferred_element_type=jnp.float32)
    d2 = sqi + jnp.transpose(sqj) - 2.0 * dot
    o_ref[...] = jnp.sqrt(jnp.clip(d2, 0.0, None) + 1e-12)


def _pairwise_dists(X, tm=512):
    n, d = X.shape
    tm = min(tm, n)
    return pl.pallas_call(
        _dist_body,
        grid=(n // tm,),
        in_specs=[
            pl.BlockSpec((tm, d), lambda i: (i, 0)),
            pl.BlockSpec((n, d), lambda i: (0, 0)),
        ],
        out_specs=pl.BlockSpec((tm, n), lambda i: (i, 0)),
        out_shape=jax.ShapeDtypeStruct((n, n), jnp.float32),
        interpret=_INTERPRET,
    )(X, X)


def _stats_body(bnd_ref, cs_ref, d_ref, y_ref, yc_ref, o_ref, *, tr, w):
    i = pl.program_id(0)
    sk = pl.multiple_of(cs_ref[0, i], 128)
    dw = d_ref[:, pl.ds(sk, w)]
    yw = y_ref[:, pl.ds(sk, w)]
    yr = yc_ref[...]
    c_lo = bnd_ref[0, i]
    c_hi = bnd_ref[1, i]
    lane = jax.lax.broadcasted_iota(jnp.int32, (1, 128), 1)

    @pl.when(i == 0)
    def _init():
        row = jax.lax.broadcasted_iota(jnp.int32, (8, 128), 0)
        o_ref[...] = jnp.where(row == 0, jnp.inf,
                               jnp.where(row == 1, -jnp.inf, 0.0)
                               ).astype(jnp.float32)

    for c in range(_NC):
        @pl.when(jnp.logical_and(c >= c_lo, c <= c_hi))
        def _acc(c=c):
            m2 = jnp.logical_and(yr == c, yw == c)
            dmin_c = jnp.min(jnp.where(m2, dw, jnp.inf))
            dmax_c = jnp.max(jnp.where(m2, dw, -jnp.inf))
            p0 = o_ref[0:1, :]
            o_ref[0:1, :] = jnp.where(lane == c,
                                      jnp.minimum(p0, dmin_c), p0)
            p1 = o_ref[1:2, :]
            o_ref[1:2, :] = jnp.where(lane == c,
                                      jnp.maximum(p1, dmax_c), p1)


def _class_stats(dists, y2, ycol, bounds, cstarts, w, tr=128):
    n = dists.shape[0]
    return pl.pallas_call(
        functools.partial(_stats_body, tr=tr, w=w),
        grid=(n // tr,),
        in_specs=[
            pl.BlockSpec(memory_space=pltpu.SMEM),
            pl.BlockSpec(memory_space=pltpu.SMEM),
            pl.BlockSpec((tr, n), lambda i: (i, 0)),
            pl.BlockSpec((1, n), lambda i: (0, 0)),
            pl.BlockSpec((tr, 1), lambda i: (i, 0)),
        ],
        out_specs=pl.BlockSpec((8, 128), lambda i: (0, 0)),
        out_shape=jax.ShapeDtypeStruct((8, 128), jnp.float32),
        interpret=_INTERPRET,
    )(bounds, cstarts, dists, y2, ycol)


def _sinkhorn_body(p_ref, cs_ref, d_ref, y_ref, yc_ref, o_ref, *, tr, w):
    i = pl.program_id(0)
    sk = pl.multiple_of(cs_ref[0, i], 128)
    dw = d_ref[:, pl.ds(sk, w)]
    yw = y_ref[:, pl.ds(sk, w)]
    yr = yc_ref[...]
    mask = yw == yr

    def sel(row):
        v = jnp.zeros((tr, 1), jnp.float32)
        for c in range(_NC):
            v = jnp.where(yr == c, p_ref[row, c], v)
        return v

    smin = sel(0)
    inv_den = sel(1)
    log_nu0 = sel(2)
    log_nu1 = sel(3)
    log_mu = sel(4)

    inv_eps = 1.0 / _EPS
    s = jnp.log(1.0 / (dw + 1e-6))
    sn = (s - smin) * inv_den
    c0i = sn * sn * inv_eps
    c1i = (sn - 1.0) * (sn - 1.0) * inv_eps
    f0 = jnp.where(mask, 0.0, -jnp.inf)

    def body(_, carry):
        fi, _g0, _g1 = carry
        t0 = fi - c0i
        t1 = fi - c1i
        m0 = jnp.max(t0, axis=1, keepdims=True)
        m1 = jnp.max(t1, axis=1, keepdims=True)
        g0i = log_nu0 - (m0 + jnp.log(jnp.sum(jnp.exp(t0 - m0), axis=1,
                                              keepdims=True)))
        g1i = log_nu1 - (m1 + jnp.log(jnp.sum(jnp.exp(t1 - m1), axis=1,
                                              keepdims=True)))
        u0 = g0i - c0i
        u1 = g1i - c1i
        mm = jnp.maximum(u0, u1)
        lse = mm + jnp.log1p(jnp.exp(-jnp.abs(u1 - u0)))
        fi = log_mu - lse
        fi = jnp.where(mask, fi, -jnp.inf)
        return fi, g0i, g1i

    zero = jnp.zeros((tr, 1), jnp.float32)
    fi, _, g1i = jax.lax.fori_loop(0, _MAX_ITER, body, (f0, zero, zero))

    wgt = jnp.exp(fi + g1i - c1i)
    anchor = jnp.sum(jnp.where(mask, dw * wgt, 0.0), axis=1, keepdims=True)
    hard = jnp.where(anchor - d_ref[...] >= 0.0, 1.0, 0.0)
    o_ref[...] = jnp.sum(hard, axis=1, keepdims=True)


def _sinkhorn_counts(dists, y2, ycol, params, cstarts, w, tr=128):
    n = dists.shape[0]
    return pl.pallas_call(
        functools.partial(_sinkhorn_body, tr=tr, w=w),
        grid=(n // tr,),
        in_specs=[
            pl.BlockSpec(memory_space=pltpu.SMEM),
            pl.BlockSpec(memory_space=pltpu.SMEM),
            pl.BlockSpec((tr, n), lambda i: (i, 0)),
            pl.BlockSpec((1, n), lambda i: (0, 0)),
            pl.BlockSpec((tr, 1), lambda i: (i, 0)),
        ],
        out_specs=pl.BlockSpec((tr, 1), lambda i: (i, 0)),
        out_shape=jax.ShapeDtypeStruct((n, 1), jnp.float32),
        interpret=_INTERPRET,
    )(params, cstarts, dists, y2, ycol)


def kernel(X, y):
    n = X.shape[0]
    tr = min(256, n)
    w = min(768, n)

    oh = (y[:, None] == jnp.arange(_NC, dtype=y.dtype)[None, :]
          ).astype(jnp.int32)
    cnt_i = jnp.sum(oh, axis=0)
    cnt = cnt_i.astype(jnp.float32)
    starts = jnp.concatenate([jnp.zeros((1,), jnp.int32),
                              jnp.cumsum(cnt_i)[:-1]])
    ends = starts + cnt_i
    within = jnp.cumsum(oh, axis=0) - oh
    rank = starts[y] + jnp.sum(within * oh, axis=1)
    perm = jnp.zeros((n,), jnp.int32).at[rank].set(
        jnp.arange(n, dtype=jnp.int32))
    yp = y[perm]
    Xp = X[perm]
    y2 = jnp.reshape(yp, (1, n))
    ycol = jnp.reshape(yp, (n, 1))
    c_lo = yp[0::tr].astype(jnp.int32)
    c_hi = yp[tr - 1::tr].astype(jnp.int32)
    win_lo = starts[c_lo]
    win_hi = ends[c_hi]
    cstart = jnp.clip((win_lo // 128) * 128, 0, n - w)
    fits = jnp.all(win_hi - cstart <= w)
    nb = n // tr
    cstarts = jnp.reshape(cstart, (1, nb))
    zeros = jnp.zeros_like(cstarts)
    bounds = jnp.stack([c_lo, c_hi])
    bounds_full = jnp.stack([jnp.zeros((nb,), jnp.int32),
                             jnp.full((nb,), _NC - 1, jnp.int32)])

    dists = _pairwise_dists(Xp)

    stats = jax.lax.cond(
        fits,
        lambda: _class_stats(dists, y2, ycol, bounds, cstarts, w, tr),
        lambda: _class_stats(dists, y2, ycol, bounds_full, zeros, n, tr),
    )

    dmin = stats[0, :_NC]
    dmax = stats[1, :_NC]

    smax = jnp.log(1.0 / (dmin + 1e-6))
    smin = jnp.log(1.0 / (dmax + 1e-6))
    inv_den = 1.0 / (smax - smin + 1e-8)
    kk = float(_K + 1)
    log_nu0 = jnp.log((cnt - kk) / cnt)
    log_nu1 = jnp.log(kk / cnt)
    log_mu = -jnp.log(cnt)

    params = jnp.zeros((8, 16), jnp.float32)
    params = params.at[0, :_NC].set(smin)
    params = params.at[1, :_NC].set(inv_den)
    params = params.at[2, :_NC].set(log_nu0)
    params = params.at[3, :_NC].set(log_nu1)
    params = params.at[4, :_NC].set(log_mu)

    cnts = jax.lax.cond(
        fits,
        lambda: _sinkhorn_counts(dists, y2, ycol, params, cstarts, w, tr),
        lambda: _sinkhorn_counts(dists, y2, ycol, params, zeros, n, tr),
    )[:, 0]

    m_shift = jnp.float32(-1.0) + jnp.float32(1e-7)
    avg_m_i = jnp.mean(digamma(cnts + m_shift))
    n_x_w = cnt / float(n)
    avg_n_x = jnp.sum(n_x_w * digamma(cnt))
    mi = (digamma(jnp.asarray(float(n), jnp.float32)) - avg_n_x
          + digamma(jnp.asarray(float(_K), jnp.float32)) - avg_m_i)
    mi = mi / jnp.log(jnp.asarray(2.0, jnp.float32))
    return jax.nn.relu(mi)
